# baseline (speedup 1.0000x reference)
import functools

import jax
import jax.numpy as jnp
from jax import lax
from jax.experimental import pallas as pl
from jax.experimental.pallas import tpu as pltpu
from jax.experimental.pallas import tpu_sc as plsc

N = 10000
E = 320000
D = 128
BIG = 32
HLAT = 2

NC = 2
NS = 16
NW = NC * NS
RG = 125
NROWS = E // RG
NHALF = NROWS // 2
NPT = N // NS
NBUF = 8

_mesh = plsc.VectorSubcoreMesh(
    core_axis_name="c", subcore_axis_name="s", num_cores=NC, num_subcores=NS
)

_sc_params = pltpu.CompilerParams(
    use_tc_tiling_on_sc=False, skip_device_barrier=True
)


def _make_gather(nrows):
    rpw = nrows // NW
    group = rpw // NBUF
    scratch = (
        [pltpu.VMEM((rpw, RG), jnp.int32)] * 2
        + [pltpu.VMEM((RG, BIG), jnp.float32)] * (2 * NBUF)
        + [pltpu.SemaphoreType.DMA] * (2 * NBUF)
    )

    @functools.partial(
        pl.kernel,
        out_type=jax.ShapeDtypeStruct((nrows, RG, BIG), jnp.float32),
        mesh=_mesh,
        compiler_params=_sc_params,
        scratch_types=scratch,
    )
    def _gather(p_hbm, q_hbm, dst_hbm, src_hbm, h_hbm, ds_all, sr_all, *bufs):
        av = bufs[0:NBUF]
        bv = bufs[NBUF:2 * NBUF]
        gsem = bufs[2 * NBUF:3 * NBUF]
        wsem = bufs[3 * NBUF:4 * NBUF]
        wid = lax.axis_index("s") * NC + lax.axis_index("c")
        base = wid * rpw
        pltpu.sync_copy(dst_hbm.at[pl.ds(base, rpw)], ds_all)
        pltpu.sync_copy(src_hbm.at[pl.ds(base, rpw)], sr_all)

        def fire(row, i):
            pltpu.async_copy(p_hbm.at[ds_all.at[row]], av[i], gsem[i])
            pltpu.async_copy(q_hbm.at[sr_all.at[row]], bv[i], gsem[i])

        def drain_gather(i):
            pltpu.make_async_copy(h_hbm.at[0], av[i], gsem[i]).wait()
            pltpu.make_async_copy(h_hbm.at[0], av[i], gsem[i]).wait()

        def drain_write(i):
            pltpu.make_async_copy(av[i], h_hbm.at[0], wsem[i]).wait()

        def add_rows(i):
            a, b = av[i], bv[i]

            @plsc.parallel_loop(0, RG, unroll=8)
            def _(k):
                a[k, 0:16] = a[k, 0:16] + b[k, 0:16]
                a[k, 16:32] = a[k, 16:32] + b[k, 16:32]

        for i in range(NBUF):
            fire(i, i)

        def body(u, carry):
            r0 = NBUF * u
            for i in range(NBUF):
                drain_gather(i)
                add_rows(i)
                pltpu.async_copy(av[i], h_hbm.at[base + r0 + i], wsem[i])

            @pl.when(u < group - 1)
            def _():
                for i in range(NBUF):
                    drain_write(i)
                    fire(r0 + NBUF + i, i)

            return carry

        lax.fori_loop(0, group, body, 0)
        for i in range(NBUF):
            drain_write(i)

    return _gather


_sc_gather = _make_gather(NHALF)


def _scatter_pipeline(m_hbm, accm, accc, ds_all, m0, m1, ones_v,
                      s0, s1, sa0, sa1, base, rpw):
    def drain_load(mv, sem):
        pltpu.make_async_copy(m_hbm.at[0], mv, sem).wait()

    def drain_add(mv, sem):
        pltpu.make_async_copy(mv, accm.at[ds_all.at[0]], sem).wait()

    pltpu.async_copy(m_hbm.at[base], m0, s0)

    def body(t, carry):
        r0 = 2 * t

        @pl.when(t > 0)
        def _():
            drain_add(m1, sa1)

        pltpu.async_copy(m_hbm.at[base + r0 + 1], m1, s1)
        drain_load(m0, s0)
        pltpu.async_copy(m0, accm.at[ds_all.at[r0]], sa0, add=True)
        if accc is not None:
            pltpu.sync_copy(ones_v, accc.at[ds_all.at[r0]], add=True)

        @pl.when(t < rpw // 2 - 1)
        def _():
            drain_add(m0, sa0)
            pltpu.async_copy(m_hbm.at[base + r0 + 2], m0, s0)

        drain_load(m1, s1)
        pltpu.async_copy(m1, accm.at[ds_all.at[r0 + 1]], sa1, add=True)
        if accc is not None:
            pltpu.sync_copy(ones_v, accc.at[ds_all.at[r0 + 1]], add=True)
        return carry

    lax.fori_loop(0, rpw // 2, body, 0)
    drain_add(m0, sa0)
    drain_add(m1, sa1)


def _make_scatter(nrows, with_cnt):
    rpw = nrows // NW
    out_type = [jax.ShapeDtypeStruct((NC, N, BIG), jnp.float32)]
    scratch = [
        pltpu.VMEM((rpw, RG), jnp.int32),
        pltpu.VMEM((RG, BIG), jnp.float32),
        pltpu.VMEM((RG, BIG), jnp.float32),
        pltpu.VMEM_SHARED((N, BIG), jnp.float32),
        pltpu.SemaphoreType.DMA,
        pltpu.SemaphoreType.DMA,
        pltpu.SemaphoreType.DMA,
        pltpu.SemaphoreType.DMA,
    ]
    if with_cnt:
        out_type.append(jax.ShapeDtypeStruct((NC, N, 16), jnp.float32))
        scratch += [
            pltpu.VMEM((RG, 16), jnp.float32),
            pltpu.VMEM_SHARED((N, 16), jnp.float32),
        ]

    def _body(m_hbm, dst_hbm, z32_hbm, *rest):
        if with_cnt:
            (z16_hbm, ones_hbm, sm_hbm, sc_hbm,
             ds_all, m0, m1, accm, s0, s1, sa0, sa1, ones_v, accc) = rest
        else:
            (sm_hbm, ds_all, m0, m1, accm, s0, s1, sa0, sa1) = rest
            ones_v = accc = sc_hbm = z16_hbm = ones_hbm = None
        cid = lax.axis_index("c")
        sid = lax.axis_index("s")
        wid = sid * NC + cid
        base = wid * rpw
        t0 = sid * NPT
        pltpu.sync_copy(dst_hbm.at[pl.ds(base, rpw)], ds_all)
        pltpu.sync_copy(z32_hbm.at[pl.ds(t0, NPT)], accm.at[pl.ds(t0, NPT)])
        if with_cnt:
            pltpu.sync_copy(z16_hbm.at[pl.ds(t0, NPT)], accc.at[pl.ds(t0, NPT)])
            pltpu.sync_copy(ones_hbm, ones_v)
        plsc.subcore_barrier()

        _scatter_pipeline(m_hbm, accm, accc, ds_all, m0, m1, ones_v,
                          s0, s1, sa0, sa1, base, rpw)

        plsc.subcore_barrier()
        pltpu.sync_copy(accm.at[pl.ds(t0, NPT)], sm_hbm.at[cid, pl.ds(t0, NPT)])
        if with_cnt:
            pltpu.sync_copy(accc.at[pl.ds(t0, NPT)], sc_hbm.at[cid, pl.ds(t0, NPT)])

    return functools.partial(
        pl.kernel,
        out_type=tuple(out_type) if with_cnt else out_type[0],
        mesh=_mesh,
        compiler_params=_sc_params,
        scratch_types=scratch,
    )(_body)


_sc_scatter_cnt = _make_scatter(NHALF, True)
_sc_scatter = _make_scatter(NHALF, False)


def _tc_prep_body(x_ref, g_ref, bt_ref, w1d_ref, w1b_ref, b1_ref, p_ref, q_ref):
    x = x_ref[...]
    mean = jnp.mean(x, axis=0, keepdims=True)
    xc = x - mean
    var = jnp.mean(xc * xc, axis=0, keepdims=True)
    xn = xc * lax.rsqrt(var + 1e-5) * g_ref[...] + bt_ref[...]
    p_ref[...] = (
        jnp.dot(xn, w1d_ref[...], preferred_element_type=jnp.float32) + b1_ref[...]
    )
    q_ref[...] = jnp.dot(xn, w1b_ref[...], preferred_element_type=jnp.float32)


_tc_prep = pl.pallas_call(
    _tc_prep_body,
    out_shape=(
        jax.ShapeDtypeStruct((N, BIG), jnp.float32),
        jax.ShapeDtypeStruct((N, BIG), jnp.float32),
    ),
)


EPACK = NHALF * RG * BIG // D
BE = 4000


def _tc_mlp_body(h_ref, w_ref, b_ref, m_ref):
    h = jnp.maximum(h_ref[...], 0.0)
    m = jnp.dot(h, w_ref[...], preferred_element_type=jnp.float32) + b_ref[...]
    m_ref[...] = jnp.maximum(m, 0.0)


_tc_mlp = pl.pallas_call(
    _tc_mlp_body,
    grid=(EPACK // BE,),
    in_specs=[
        pl.BlockSpec((BE, D), lambda i: (i, 0)),
        pl.BlockSpec((D, D), lambda i: (0, 0)),
        pl.BlockSpec((1, D), lambda i: (0, 0)),
    ],
    out_specs=pl.BlockSpec((BE, D), lambda i: (i, 0)),
    out_shape=jax.ShapeDtypeStruct((EPACK, D), jnp.float32),
)


def _tc_mid_body(sma_ref, smb_ref, sca_ref, scb_ref, eps_ref,
                 wmu_ref, bmu_ref, wvar_ref, bvar_ref,
                 wd1d_ref, wd1b_ref, bd1_ref,
                 mu_ref, lv_ref, r_ref, s_ref, cnt_ref):
    s = sma_ref[0] + sma_ref[1] + smb_ref[0] + smb_ref[1]
    cnt = (sca_ref[0, :, 0:1] + sca_ref[1, :, 0:1]
           + scb_ref[0, :, 0:1] + scb_ref[1, :, 0:1])
    henc = s / jnp.maximum(cnt, 1.0)
    mu = jnp.dot(henc, wmu_ref[...], preferred_element_type=jnp.float32) + bmu_ref[...]
    lv = jnp.dot(henc, wvar_ref[...], preferred_element_type=jnp.float32) + bvar_ref[...]
    z = mu + eps_ref[...] * jnp.exp(0.5 * lv)
    z0 = z[:, 0:1]
    z1 = z[:, 1:2]
    r_ref[...] = z0 * wd1d_ref[0:1, :] + z1 * wd1d_ref[1:2, :] + bd1_ref[...]
    s_ref[...] = z0 * wd1b_ref[0:1, :] + z1 * wd1b_ref[1:2, :]
    mu_ref[...] = mu
    lv_ref[...] = lv
    cnt_ref[...] = cnt


BN = 2000

_tc_mid = pl.pallas_call(
    _tc_mid_body,
    grid=(N // BN,),
    in_specs=[
        pl.BlockSpec((NC, BN, BIG), lambda i: (0, i, 0)),
        pl.BlockSpec((NC, BN, BIG), lambda i: (0, i, 0)),
        pl.BlockSpec((NC, BN, 16), lambda i: (0, i, 0)),
        pl.BlockSpec((NC, BN, 16), lambda i: (0, i, 0)),
        pl.BlockSpec((BN, HLAT), lambda i: (i, 0)),
        pl.BlockSpec((BIG, HLAT), lambda i: (0, 0)),
        pl.BlockSpec((1, HLAT), lambda i: (0, 0)),
        pl.BlockSpec((BIG, HLAT), lambda i: (0, 0)),
        pl.BlockSpec((1, HLAT), lambda i: (0, 0)),
        pl.BlockSpec((HLAT, BIG), lambda i: (0, 0)),
        pl.BlockSpec((HLAT, BIG), lambda i: (0, 0)),
        pl.BlockSpec((1, BIG), lambda i: (0, 0)),
    ],
    out_specs=(
        pl.BlockSpec((BN, HLAT), lambda i: (i, 0)),
        pl.BlockSpec((BN, HLAT), lambda i: (i, 0)),
        pl.BlockSpec((BN, BIG), lambda i: (i, 0)),
        pl.BlockSpec((BN, BIG), lambda i: (i, 0)),
        pl.BlockSpec((BN, 1), lambda i: (i, 0)),
    ),
    out_shape=(
        jax.ShapeDtypeStruct((N, HLAT), jnp.float32),
        jax.ShapeDtypeStruct((N, HLAT), jnp.float32),
        jax.ShapeDtypeStruct((N, BIG), jnp.float32),
        jax.ShapeDtypeStruct((N, BIG), jnp.float32),
        jax.ShapeDtypeStruct((N, 1), jnp.float32),
    ),
)


def _tc_out_body(ta_ref, tb_ref, cnt_ref, wd3_ref, bd3_ref, out_ref):
    t = ta_ref[0] + ta_ref[1] + tb_ref[0] + tb_ref[1]
    cnt = cnt_ref[...]
    tmean = t / jnp.maximum(cnt, 1.0)
    mask = jnp.where(cnt > 0, 1.0, 0.0)
    out_ref[...] = (
        jnp.dot(tmean, wd3_ref[...], preferred_element_type=jnp.float32)
        + bd3_ref[...] * mask
    )


_tc_out = pl.pallas_call(
    _tc_out_body,
    grid=(N // BN,),
    in_specs=[
        pl.BlockSpec((NC, BN, BIG), lambda i: (0, i, 0)),
        pl.BlockSpec((NC, BN, BIG), lambda i: (0, i, 0)),
        pl.BlockSpec((BN, 1), lambda i: (i, 0)),
        pl.BlockSpec((BIG, D), lambda i: (0, 0)),
        pl.BlockSpec((1, D), lambda i: (0, 0)),
    ],
    out_specs=pl.BlockSpec((BN, D), lambda i: (i, 0)),
    out_shape=jax.ShapeDtypeStruct((N, D), jnp.float32),
)


def kernel(x, edge_index, eps, gamma, beta, W1, b1, W2, b2, Wmu, bmu, Wvar, bvar,
           Wd1, bd1, Wd2, bd2, Wd3, bd3):
    src = edge_index[0].reshape(NROWS, RG)
    dst = edge_index[1].reshape(NROWS, RG)
    srcs = (src[:NHALF], src[NHALF:])
    dsts = (dst[:NHALF], dst[NHALF:])
    w1a, w1b = W1[:D], W1[D:]

    p_tab, q_tab = _tc_prep(
        x, gamma.reshape(1, D), beta.reshape(1, D), w1a - w1b, w1b, b1.reshape(1, BIG)
    )

    eye4 = jnp.eye(4, dtype=jnp.float32)
    w2blk = jnp.kron(eye4, W2)
    b2t = jnp.tile(b2, 4).reshape(1, D)
    z32 = jnp.zeros((N, BIG), jnp.float32)
    z16 = jnp.zeros((N, 16), jnp.float32)
    ones16 = jnp.ones((RG, 16), jnp.float32)

    sms, scs = [], []
    for hf in range(2):
        h = _sc_gather(p_tab, q_tab, dsts[hf], srcs[hf])
        m = _tc_mlp(h.reshape(EPACK, D), w2blk, b2t)
        sm, sc = _sc_scatter_cnt(m.reshape(NHALF, RG, BIG), dsts[hf], z32, z16, ones16)
        sms.append(sm)
        scs.append(sc)

    wd1a, wd1b = Wd1[:HLAT], Wd1[HLAT:]
    mu, lv, r_tab, s_tab, cnt = _tc_mid(
        sms[0], sms[1], scs[0], scs[1], eps,
        Wmu, bmu.reshape(1, HLAT), Wvar, bvar.reshape(1, HLAT),
        wd1a - wd1b, wd1b, bd1.reshape(1, BIG)
    )

    wd2blk = jnp.kron(eye4, Wd2)
    bd2t = jnp.tile(bd2, 4).reshape(1, D)
    tms = []
    for hf in range(2):
        h = _sc_gather(r_tab, s_tab, dsts[hf], srcs[hf])
        m = _tc_mlp(h.reshape(EPACK, D), wd2blk, bd2t)
        tms.append(_sc_scatter(m.reshape(NHALF, RG, BIG), dsts[hf], z32))

    out = _tc_out(tms[0], tms[1], cnt, Wd3, bd3.reshape(1, D))
    return (out, mu, lv)


# speedup vs baseline: 1.0946x; 1.0946x over previous
import functools

import jax
import jax.numpy as jnp
from jax import lax
from jax.experimental import pallas as pl
from jax.experimental.pallas import tpu as pltpu
from jax.experimental.pallas import tpu_sc as plsc

N = 10000
E = 320000
D = 128
BIG = 32
HLAT = 2

NC = 2
NS = 16
NW = NC * NS
RG = 125
NROWS = E // RG
NHALF = NROWS // 2
NPT = N // NS
NBUF = 8
NP4 = N // 4

_mesh = plsc.VectorSubcoreMesh(
    core_axis_name="c", subcore_axis_name="s", num_cores=NC, num_subcores=NS
)

_sc_params = pltpu.CompilerParams(
    use_tc_tiling_on_sc=False, skip_device_barrier=True
)


def _make_gather(half_base):
    rpw = NHALF // NW
    group = rpw // NBUF
    scratch = (
        [pltpu.VMEM((rpw, RG), jnp.int32)] * 2
        + [pltpu.VMEM((RG, BIG), jnp.float32)] * (2 * NBUF)
        + [pltpu.SemaphoreType.DMA] * (2 * NBUF)
    )

    @functools.partial(
        pl.kernel,
        out_type=jax.ShapeDtypeStruct((NHALF, RG, BIG), jnp.float32),
        mesh=_mesh,
        compiler_params=_sc_params,
        scratch_types=scratch,
    )
    def _gather(p_hbm, q_hbm, dst_hbm, src_hbm, h_hbm, ds_all, sr_all, *bufs):
        av = bufs[0:NBUF]
        bv = bufs[NBUF:2 * NBUF]
        gsem = bufs[2 * NBUF:3 * NBUF]
        wsem = bufs[3 * NBUF:4 * NBUF]
        wid = lax.axis_index("s") * NC + lax.axis_index("c")
        base = wid * rpw
        pltpu.sync_copy(dst_hbm.at[pl.ds(half_base + base, rpw)], ds_all)
        pltpu.sync_copy(src_hbm.at[pl.ds(half_base + base, rpw)], sr_all)

        def fire(row, i):
            pltpu.async_copy(p_hbm.at[ds_all.at[row]], av[i], gsem[i])
            pltpu.async_copy(q_hbm.at[sr_all.at[row]], bv[i], gsem[i])

        def drain_gather(i):
            pltpu.make_async_copy(h_hbm.at[0], av[i], gsem[i]).wait()
            pltpu.make_async_copy(h_hbm.at[0], av[i], gsem[i]).wait()

        def drain_write(i):
            pltpu.make_async_copy(av[i], h_hbm.at[0], wsem[i]).wait()

        def add_rows(i):
            a, b = av[i], bv[i]

            @plsc.parallel_loop(0, RG, unroll=8)
            def _(k):
                a[k, 0:16] = a[k, 0:16] + b[k, 0:16]
                a[k, 16:32] = a[k, 16:32] + b[k, 16:32]

        for i in range(NBUF):
            fire(i, i)

        def body(u, carry):
            r0 = NBUF * u
            for i in range(NBUF):
                drain_gather(i)
                add_rows(i)
                pltpu.async_copy(av[i], h_hbm.at[base + r0 + i], wsem[i])

            @pl.when(u < group - 1)
            def _():
                for i in range(NBUF):
                    drain_write(i)
                    fire(r0 + NBUF + i, i)

            return carry

        lax.fori_loop(0, group, body, 0)
        for i in range(NBUF):
            drain_write(i)

    return _gather


_sc_gather = (_make_gather(0), _make_gather(NHALF))


def _scatter_pipeline(m_hbm, accm, accc, ds_all, m0, m1, ones_v,
                      s0, s1, sa0, sa1, base, rpw):
    def drain_load(mv, sem):
        pltpu.make_async_copy(m_hbm.at[0], mv, sem).wait()

    def drain_add(mv, sem):
        pltpu.make_async_copy(mv, accm.at[ds_all.at[0]], sem).wait()

    pltpu.async_copy(m_hbm.at[base], m0, s0)

    def body(t, carry):
        r0 = 2 * t

        @pl.when(t > 0)
        def _():
            drain_add(m1, sa1)

        pltpu.async_copy(m_hbm.at[base + r0 + 1], m1, s1)
        drain_load(m0, s0)
        pltpu.async_copy(m0, accm.at[ds_all.at[r0]], sa0, add=True)
        if accc is not None:
            pltpu.sync_copy(ones_v, accc.at[ds_all.at[r0]], add=True)

        @pl.when(t < rpw // 2 - 1)
        def _():
            drain_add(m0, sa0)
            pltpu.async_copy(m_hbm.at[base + r0 + 2], m0, s0)

        drain_load(m1, s1)
        pltpu.async_copy(m1, accm.at[ds_all.at[r0 + 1]], sa1, add=True)
        if accc is not None:
            pltpu.sync_copy(ones_v, accc.at[ds_all.at[r0 + 1]], add=True)
        return carry

    lax.fori_loop(0, rpw // 2, body, 0)
    drain_add(m0, sa0)
    drain_add(m1, sa1)


def _make_scatter(half_base, with_cnt):
    rpw = NHALF // NW
    out_type = [jax.ShapeDtypeStruct((NC, N, BIG), jnp.float32)]
    scratch = [
        pltpu.VMEM((rpw, RG), jnp.int32),
        pltpu.VMEM((RG, BIG), jnp.float32),
        pltpu.VMEM((RG, BIG), jnp.float32),
        pltpu.VMEM_SHARED((N, BIG), jnp.float32),
        pltpu.SemaphoreType.DMA,
        pltpu.SemaphoreType.DMA,
        pltpu.SemaphoreType.DMA,
        pltpu.SemaphoreType.DMA,
    ]
    if with_cnt:
        out_type.append(jax.ShapeDtypeStruct((NC, N, BIG), jnp.float32))
        scratch += [
            pltpu.VMEM((RG, BIG), jnp.float32),
            pltpu.VMEM_SHARED((N, BIG), jnp.float32),
        ]

    def _body(m_hbm, dst_hbm, z32_hbm, *rest):
        if with_cnt:
            (ones_hbm, sm_hbm, sc_hbm,
             ds_all, m0, m1, accm, s0, s1, sa0, sa1, ones_v, accc) = rest
        else:
            (sm_hbm, ds_all, m0, m1, accm, s0, s1, sa0, sa1) = rest
            ones_v = accc = sc_hbm = ones_hbm = None
        cid = lax.axis_index("c")
        sid = lax.axis_index("s")
        wid = sid * NC + cid
        base = wid * rpw
        t0 = sid * NPT
        pltpu.sync_copy(dst_hbm.at[pl.ds(half_base + base, rpw)], ds_all)
        pltpu.sync_copy(z32_hbm.at[pl.ds(t0, NPT)], accm.at[pl.ds(t0, NPT)])
        if with_cnt:
            pltpu.sync_copy(z32_hbm.at[pl.ds(t0, NPT)], accc.at[pl.ds(t0, NPT)])
            pltpu.sync_copy(ones_hbm, ones_v)
        plsc.subcore_barrier()

        _scatter_pipeline(m_hbm, accm, accc, ds_all, m0, m1, ones_v,
                          s0, s1, sa0, sa1, base, rpw)

        plsc.subcore_barrier()
        pltpu.sync_copy(accm.at[pl.ds(t0, NPT)], sm_hbm.at[cid, pl.ds(t0, NPT)])
        if with_cnt:
            pltpu.sync_copy(accc.at[pl.ds(t0, NPT)], sc_hbm.at[cid, pl.ds(t0, NPT)])

    return functools.partial(
        pl.kernel,
        out_type=tuple(out_type) if with_cnt else out_type[0],
        mesh=_mesh,
        compiler_params=_sc_params,
        scratch_types=scratch,
    )(_body)


_sc_scatter_cnt = (_make_scatter(0, True), _make_scatter(NHALF, True))
_sc_scatter = (_make_scatter(0, False), _make_scatter(NHALF, False))


def _tc_prep_body(x_ref, g_ref, bt_ref, w1d_ref, w1b_ref, b1_ref, p_ref, q_ref):
    def chunk_mean(v4):
        m = (v4[:, 0:D] + v4[:, D:2 * D] + v4[:, 2 * D:3 * D]
             + v4[:, 3 * D:4 * D]) * 0.25
        return jnp.concatenate([m] * 4, axis=1)

    x = x_ref[...]
    mean4 = chunk_mean(jnp.mean(x, axis=0, keepdims=True))
    xc = x - mean4
    var4 = chunk_mean(jnp.mean(xc * xc, axis=0, keepdims=True))
    xn = xc * lax.rsqrt(var4 + 1e-5) * g_ref[...] + bt_ref[...]
    p_ref[...] = (
        jnp.dot(xn, w1d_ref[...], preferred_element_type=jnp.float32) + b1_ref[...]
    )
    q_ref[...] = jnp.dot(xn, w1b_ref[...], preferred_element_type=jnp.float32)


_tc_prep = pl.pallas_call(
    _tc_prep_body,
    out_shape=(
        jax.ShapeDtypeStruct((NP4, D), jnp.float32),
        jax.ShapeDtypeStruct((NP4, D), jnp.float32),
    ),
)


EPACK = NHALF * RG * BIG // D
BE = 4000


def _tc_mlp_body(h_ref, w_ref, b_ref, m_ref):
    h = jnp.maximum(h_ref[...], 0.0)
    m = jnp.dot(h, w_ref[...], preferred_element_type=jnp.float32) + b_ref[...]
    m_ref[...] = jnp.maximum(m, 0.0)


_tc_mlp = pl.pallas_call(
    _tc_mlp_body,
    grid=(EPACK // BE,),
    in_specs=[
        pl.BlockSpec((BE, D), lambda i: (i, 0)),
        pl.BlockSpec((D, D), lambda i: (0, 0)),
        pl.BlockSpec((1, D), lambda i: (0, 0)),
    ],
    out_specs=pl.BlockSpec((BE, D), lambda i: (i, 0)),
    out_shape=jax.ShapeDtypeStruct((EPACK, D), jnp.float32),
)


def _tc_mid_body(sma_ref, smb_ref, sca_ref, scb_ref, eps_ref,
                 wmu_ref, bmu_ref, wvar_ref, bvar_ref,
                 wd1d_ref, wd1b_ref, bd1_ref,
                 mu_ref, lv_ref, r_ref, s_ref, cnt_ref):
    s = sma_ref[0] + sma_ref[1] + smb_ref[0] + smb_ref[1]
    cnt = sca_ref[0] + sca_ref[1] + scb_ref[0] + scb_ref[1]
    henc = s / jnp.maximum(cnt, 1.0)
    mu = jnp.dot(henc, wmu_ref[...], preferred_element_type=jnp.float32) + bmu_ref[...]
    lv = jnp.dot(henc, wvar_ref[...], preferred_element_type=jnp.float32) + bvar_ref[...]
    z = mu + eps_ref[...] * jnp.exp(0.5 * lv)
    r_ref[...] = (
        jnp.dot(z, wd1d_ref[...], preferred_element_type=jnp.float32) + bd1_ref[...]
    )
    s_ref[...] = jnp.dot(z, wd1b_ref[...], preferred_element_type=jnp.float32)
    mu_ref[...] = mu
    lv_ref[...] = lv
    cnt_ref[...] = cnt


_tc_mid = pl.pallas_call(
    _tc_mid_body,
    out_shape=(
        jax.ShapeDtypeStruct((NP4, 4 * HLAT), jnp.float32),
        jax.ShapeDtypeStruct((NP4, 4 * HLAT), jnp.float32),
        jax.ShapeDtypeStruct((NP4, D), jnp.float32),
        jax.ShapeDtypeStruct((NP4, D), jnp.float32),
        jax.ShapeDtypeStruct((NP4, D), jnp.float32),
    ),
)


def _tc_out_body(ta_ref, tb_ref, cnt_ref, wd3_ref, bmask_ref, out_ref):
    t = ta_ref[0] + ta_ref[1] + tb_ref[0] + tb_ref[1]
    cnt = cnt_ref[...]
    tmean = t / jnp.maximum(cnt, 1.0)
    flag = jnp.minimum(cnt, 1.0)
    out_ref[...] = (
        jnp.dot(tmean, wd3_ref[...], preferred_element_type=jnp.float32)
        + jnp.dot(flag, bmask_ref[...], preferred_element_type=jnp.float32)
    )


_tc_out = pl.pallas_call(
    _tc_out_body,
    out_shape=jax.ShapeDtypeStruct((NP4, 4 * D), jnp.float32),
)


def _blockdiag(w):
    return jnp.kron(jnp.eye(4, dtype=jnp.float32), w)


def kernel(x, edge_index, eps, gamma, beta, W1, b1, W2, b2, Wmu, bmu, Wvar, bvar,
           Wd1, bd1, Wd2, bd2, Wd3, bd3):
    src = edge_index[0].reshape(NROWS, RG)
    dst = edge_index[1].reshape(NROWS, RG)
    w1a, w1b = W1[:D], W1[D:]

    p_tab, q_tab = _tc_prep(
        x.reshape(NP4, 4 * D),
        jnp.tile(gamma, 4).reshape(1, 4 * D),
        jnp.tile(beta, 4).reshape(1, 4 * D),
        _blockdiag(w1a - w1b), _blockdiag(w1b),
        jnp.tile(b1, 4).reshape(1, D),
    )
    p_tab = p_tab.reshape(N, BIG)
    q_tab = q_tab.reshape(N, BIG)

    w2blk = _blockdiag(W2)
    b2t = jnp.tile(b2, 4).reshape(1, D)
    z32 = jnp.zeros((N, BIG), jnp.float32)
    ones32 = jnp.ones((RG, BIG), jnp.float32)

    sms, scs = [], []
    for hf in range(2):
        h = _sc_gather[hf](p_tab, q_tab, dst, src)
        m = _tc_mlp(h.reshape(EPACK, D), w2blk, b2t)
        sm, sc = _sc_scatter_cnt[hf](m.reshape(NHALF, RG, BIG), dst, z32, ones32)
        sms.append(sm.reshape(NC, NP4, D))
        scs.append(sc.reshape(NC, NP4, D))

    wd1a, wd1b = Wd1[:HLAT], Wd1[HLAT:]
    mu_p, lv_p, r_tab, s_tab, cnt_p = _tc_mid(
        sms[0], sms[1], scs[0], scs[1], eps.reshape(NP4, 4 * HLAT),
        _blockdiag(Wmu), jnp.tile(bmu, 4).reshape(1, 4 * HLAT),
        _blockdiag(Wvar), jnp.tile(bvar, 4).reshape(1, 4 * HLAT),
        _blockdiag(wd1a - wd1b), _blockdiag(wd1b),
        jnp.tile(bd1, 4).reshape(1, D),
    )
    r_tab = r_tab.reshape(N, BIG)
    s_tab = s_tab.reshape(N, BIG)

    wd2blk = _blockdiag(Wd2)
    bd2t = jnp.tile(bd2, 4).reshape(1, D)
    tms = []
    for hf in range(2):
        h = _sc_gather[hf](r_tab, s_tab, dst, src)
        m = _tc_mlp(h.reshape(EPACK, D), wd2blk, bd2t)
        tms.append(_sc_scatter[hf](m.reshape(NHALF, RG, BIG), dst, z32)
                   .reshape(NC, NP4, D))

    bmask = _blockdiag(jnp.ones((BIG, 1), jnp.float32) / BIG * bd3[None, :])
    out_p = _tc_out(tms[0], tms[1], cnt_p, _blockdiag(Wd3), bmask)
    return (out_p.reshape(N, D), mu_p.reshape(N, HLAT), lv_p.reshape(N, HLAT))
